# Initial kernel scaffold; baseline (speedup 1.0000x reference)
import jax
import jax.numpy as jnp
from jax.experimental import pallas as pl


def kernel(x, edge_index, edge_weight, gcn1_W, gcn1_b, gcn2_W, gcn2_b, hgcn1_W, hgcn1_b, hgcn2_W, hgcn2_b, cls_W, cls_b):
    raise NotImplementedError("write your pallas kernel here")


# speedup vs baseline: 1.0526x; 1.0526x over previous
import jax
import jax.numpy as jnp
from jax.experimental import pallas as pl

N, E, D, H, C = 10000, 320000, 128, 128, 40


def _gcn_conv(x, src, dst, ew, W, b):
    n = x.shape[0]
    xw = x @ W
    loop = jnp.arange(n, dtype=src.dtype)
    s = jnp.concatenate([src, loop])
    d = jnp.concatenate([dst, loop])
    w = jnp.concatenate([ew, jnp.ones((n,), dtype=ew.dtype)])
    deg = jnp.zeros((n,), dtype=xw.dtype).at[d].add(w)
    dinv = jnp.where(deg > 0, jax.lax.rsqrt(jnp.maximum(deg, 1e-12)), 0.0)
    norm = dinv[s] * w * dinv[d]
    out = jnp.zeros_like(xw).at[d].add(norm[:, None] * xw[s])
    return out + b


def _head_kernel(hg_ref, hh_ref, w_ref, b_ref, zg_ref, zh_ref, z_ref, lg_ref):
    hg = hg_ref[...]
    hh = hh_ref[...]

    def l2n(v):
        nrm = jnp.sqrt(jnp.sum(v * v, axis=1, keepdims=True))
        return v / jnp.maximum(nrm, 1e-12)

    zg = l2n(hg)
    zh = l2n(hh)
    z = l2n(0.6 * zg + 0.4 * zh)
    zg_ref[...] = zg
    zh_ref[...] = zh
    z_ref[...] = z
    lg_ref[...] = jnp.dot(z, w_ref[...], preferred_element_type=jnp.float32) + b_ref[...]


def _head(h_gcn, h_hgcn, cls_W, cls_b):
    blk = 1000
    grid = (N // blk,)
    return pl.pallas_call(
        _head_kernel,
        grid=grid,
        in_specs=[
            pl.BlockSpec((blk, H), lambda i: (i, 0)),
            pl.BlockSpec((blk, H), lambda i: (i, 0)),
            pl.BlockSpec((H, C), lambda i: (0, 0)),
            pl.BlockSpec((C,), lambda i: (0,)),
        ],
        out_specs=[
            pl.BlockSpec((blk, H), lambda i: (i, 0)),
            pl.BlockSpec((blk, H), lambda i: (i, 0)),
            pl.BlockSpec((blk, H), lambda i: (i, 0)),
            pl.BlockSpec((blk, C), lambda i: (i, 0)),
        ],
        out_shape=[
            jax.ShapeDtypeStruct((N, H), jnp.float32),
            jax.ShapeDtypeStruct((N, H), jnp.float32),
            jax.ShapeDtypeStruct((N, H), jnp.float32),
            jax.ShapeDtypeStruct((N, C), jnp.float32),
        ],
    )(h_gcn, h_hgcn, cls_W, cls_b)


def kernel(x, edge_index, edge_weight, gcn1_W, gcn1_b, gcn2_W, gcn2_b,
           hgcn1_W, hgcn1_b, hgcn2_W, hgcn2_b, cls_W, cls_b):
    src, dst = edge_index[0], edge_index[1]
    h = jax.nn.relu(_gcn_conv(x, src, dst, edge_weight, gcn1_W, gcn1_b))
    h = _gcn_conv(h, src, dst, edge_weight, gcn2_W, gcn2_b)
    g = jax.nn.relu(_gcn_conv(x, src, dst, edge_weight, hgcn1_W, hgcn1_b))
    g = _gcn_conv(g, src, dst, edge_weight, hgcn2_W, hgcn2_b)
    z_gcn, z_hgcn, z, logits = _head(h, g, cls_W, cls_b)
    return (z_gcn, z_hgcn, z, logits)


# speedup vs baseline: 6.7669x; 6.4284x over previous
import dataclasses
import functools

import jax
import jax.numpy as jnp
from jax import lax
from jax.experimental import pallas as pl
from jax.experimental.pallas import tpu as pltpu
from jax.experimental.pallas import tpu_sc as plsc

N, E, D, H, C = 10000, 320000, 128, 128, 40
NC, NS, LN = 2, 16, 16
K = 128
E3 = E + N
NB = 168
EPT = NB * K
E3P = EPT * NS
EPT_SC = E3P // (NC * NS)
CHB = 24
RPT = N // NS
EPT_DEG = E // (NC * NS)

_f32 = jnp.float32
_i32 = jnp.int32

_vmesh = plsc.VectorSubcoreMesh(core_axis_name="c", subcore_axis_name="s")

_sc_params = pltpu.CompilerParams()
if "needs_layout_passes" in pltpu.CompilerParams.__dataclass_fields__:
    _sc_params = dataclasses.replace(_sc_params, needs_layout_passes=False)


def _deg_body(dst_hbm, w_hbm, out_hbm, acc_v, dst_v, w_v, sem):
    c = lax.axis_index("c")
    s = lax.axis_index("s")
    wid = c * NS + s
    zero16 = jnp.zeros((LN,), _f32)

    @pl.loop(0, N // LN)
    def _(i):
        acc_v[pl.ds(i * LN, LN)] = zero16

    pltpu.async_copy(dst_hbm.at[wid], dst_v, sem).wait()
    pltpu.async_copy(w_hbm.at[wid], w_v, sem).wait()

    @pl.loop(0, EPT_DEG // LN)
    def _(j):
        d16 = dst_v[pl.ds(j * LN, LN)]
        w16 = w_v[pl.ds(j * LN, LN)]
        plsc.addupdate_scatter(acc_v, [d16], w16)

    pltpu.async_copy(acc_v, out_hbm.at[wid], sem).wait()


def _sc_deg(dst_t, w_t):
    k = pl.kernel(
        _deg_body,
        out_type=jax.ShapeDtypeStruct((NC * NS, N), _f32),
        mesh=_vmesh,
        compiler_params=_sc_params,
        scratch_types=[
            pltpu.VMEM((N,), _f32),
            pltpu.VMEM((EPT_DEG,), _i32),
            pltpu.VMEM((EPT_DEG,), _f32),
            pltpu.SemaphoreType.DMA,
        ],
    )
    return k(dst_t, w_t)


def _escale_body(src_hbm, w_hbm, dinv_hbm, out_hbm, src_v, w_v, dinv_v, sem):
    c = lax.axis_index("c")
    s = lax.axis_index("s")
    wid = c * NS + s

    pltpu.async_copy(dinv_hbm.at[0], dinv_v, sem).wait()
    pltpu.async_copy(src_hbm.at[wid], src_v, sem).wait()
    pltpu.async_copy(w_hbm.at[wid], w_v, sem).wait()

    @pl.loop(0, EPT_SC // LN, unroll=4)
    def _(m):
        slc = pl.ds(m * LN, LN)
        w_v[slc] = w_v[slc] * plsc.load_gather(dinv_v, [src_v[slc]])

    pltpu.async_copy(w_v, out_hbm.at[wid], sem).wait()


def _sc_escale(src_flat, w_flat, dinv2d):
    k = pl.kernel(
        _escale_body,
        out_type=jax.ShapeDtypeStruct((NC * NS, EPT_SC), _f32),
        mesh=_vmesh,
        compiler_params=_sc_params,
        scratch_types=[
            pltpu.VMEM((EPT_SC,), _i32),
            pltpu.VMEM((EPT_SC,), _f32),
            pltpu.VMEM((N,), _f32),
            pltpu.SemaphoreType.DMA,
        ],
    )
    return k(src_flat, w_flat, dinv2d)


def _conv_body(xw_hbm, src_hbm, dst_hbm, c_hbm, out_hbm,
               acc_sh, src_v, dst_v, c_v, rows0_v, rows1_v,
               sem, gsem0, gsem1):
    c = lax.axis_index("c")
    s = lax.axis_index("s")

    zero16 = jnp.zeros((LN,), _f32)

    @pl.loop(0, K)
    def _(r):
        for cc in range(H // LN):
            rows0_v[r, pl.ds(cc * LN, LN)] = zero16

    r0 = pl.multiple_of(s * 632, 8)
    last_off = pl.multiple_of(jnp.where(s == NS - 1, 392, 504), 8)
    for off in (0, K, 2 * K, 3 * K):
        pltpu.async_copy(rows0_v, acc_sh.at[pl.ds(r0 + off, K)], sem).wait()
    pltpu.async_copy(rows0_v, acc_sh.at[pl.ds(r0 + last_off, K)], sem).wait()

    plsc.subcore_barrier()

    bufs = ((rows0_v, gsem0), (rows1_v, gsem1))

    def scale_rows(rows_v, blk):
        @pl.loop(0, K, unroll=4)
        def _(j):
            scale = plsc.load_gather(c_v, [jnp.zeros((LN,), _i32) + blk * K + j])
            for cc in range(H // LN):
                slc = (j, pl.ds(cc * LN, LN))
                rows_v[slc] = rows_v[slc] * scale

    for bstart in range(0, NB, CHB):
        pltpu.async_copy(src_hbm.at[c, s, pl.ds(bstart, CHB)], src_v, sem).wait()
        pltpu.async_copy(dst_hbm.at[s, pl.ds(bstart, CHB)], dst_v, sem).wait()
        pltpu.async_copy(c_hbm.at[s, pl.ds(bstart * K, CHB * K)], c_v, sem).wait()

        pltpu.async_copy(xw_hbm.at[src_v.at[0]], rows0_v, gsem0)
        pltpu.async_copy(xw_hbm.at[src_v.at[1]], rows1_v, gsem1)

        @pl.loop(0, CHB - 2, step=2)
        def _(b):
            for par, (rows_v, gsem) in enumerate(bufs):
                blk = b + par
                pltpu.make_async_copy(xw_hbm.at[src_v.at[0]], rows_v, gsem).wait()
                scale_rows(rows_v, blk)
                pltpu.sync_copy(rows_v, acc_sh.at[dst_v.at[blk]], add=True)
                pltpu.async_copy(xw_hbm.at[src_v.at[blk + 2]], rows_v, gsem)

        for par, (rows_v, gsem) in enumerate(bufs):
            blk = CHB - 2 + par
            pltpu.make_async_copy(xw_hbm.at[src_v.at[0]], rows_v, gsem).wait()
            scale_rows(rows_v, blk)
            pltpu.sync_copy(rows_v, acc_sh.at[dst_v.at[blk]], add=True)

    plsc.subcore_barrier()

    for off in (0, K, 2 * K, 3 * K, None):
        o = r0 + (last_off if off is None else off)
        oo = pl.multiple_of(c * N + o, 8)
        pltpu.sync_copy(acc_sh.at[pl.ds(o, K)], out_hbm.at[pl.ds(oo, K)])


def _sc_conv(xw_cat, src_t2, dst_t, c_t):
    k = pl.kernel(
        _conv_body,
        out_type=jax.ShapeDtypeStruct((2 * N, H), _f32),
        mesh=_vmesh,
        compiler_params=_sc_params,
        scratch_types=[
            pltpu.VMEM_SHARED((N, H), _f32),
            pltpu.VMEM((CHB, K), _i32),
            pltpu.VMEM((CHB, K), _i32),
            pltpu.VMEM((CHB * K,), _f32),
            pltpu.VMEM((K, H), _f32),
            pltpu.VMEM((K, H), _f32),
            pltpu.SemaphoreType.DMA,
            pltpu.SemaphoreType.DMA,
            pltpu.SemaphoreType.DMA,
        ],
    )
    return k(xw_cat, src_t2, dst_t, c_t)


_BLK = 1000
_NBLK = N // _BLK


def _mm1_kernel(x_ref, w_ref, o_ref):
    o_ref[...] = jnp.dot(x_ref[...], w_ref[0], preferred_element_type=_f32)


def _tc_mm1(x, W_stack):
    return pl.pallas_call(
        _mm1_kernel,
        grid=(2 * _NBLK,),
        in_specs=[
            pl.BlockSpec((_BLK, D), lambda i: (i % _NBLK, 0)),
            pl.BlockSpec((1, D, H), lambda i: (i // _NBLK, 0, 0)),
        ],
        out_specs=pl.BlockSpec((_BLK, H), lambda i: (i, 0)),
        out_shape=jax.ShapeDtypeStruct((2 * N, H), _f32),
    )(x, W_stack)


def _mid_kernel(a_ref, dv_ref, b_ref, w_ref, o_ref):
    h = jax.nn.relu(a_ref[...] * dv_ref[...] + b_ref[0])
    o_ref[...] = jnp.dot(h, w_ref[0], preferred_element_type=_f32)


def _tc_mid(a_cat, dinv_col, b_stack, W_stack):
    return pl.pallas_call(
        _mid_kernel,
        grid=(2 * _NBLK,),
        in_specs=[
            pl.BlockSpec((_BLK, H), lambda i: (i, 0)),
            pl.BlockSpec((_BLK, 1), lambda i: (i % _NBLK, 0)),
            pl.BlockSpec((1, 1, H), lambda i: (i // _NBLK, 0, 0)),
            pl.BlockSpec((1, H, H), lambda i: (i // _NBLK, 0, 0)),
        ],
        out_specs=pl.BlockSpec((_BLK, H), lambda i: (i, 0)),
        out_shape=jax.ShapeDtypeStruct((2 * N, H), _f32),
    )(a_cat, dinv_col, b_stack.reshape(2, 1, H), W_stack)


def _head_kernel(ag_ref, ah_ref, dv_ref, b_ref, w_ref, cb_ref,
                 zg_ref, zh_ref, z_ref, lg_ref):
    dv = dv_ref[...]
    hg = ag_ref[...] * dv + b_ref[0, :][None, :]
    hh = ah_ref[...] * dv + b_ref[1, :][None, :]

    def l2n(v):
        nrm = jnp.sqrt(jnp.sum(v * v, axis=1, keepdims=True))
        return v / jnp.maximum(nrm, 1e-12)

    zg = l2n(hg)
    zh = l2n(hh)
    z = l2n(0.6 * zg + 0.4 * zh)
    zg_ref[...] = zg
    zh_ref[...] = zh
    z_ref[...] = z
    lg_ref[...] = jnp.dot(z, w_ref[...], preferred_element_type=_f32) + cb_ref[...]


def _tc_head(a_cat, dinv_col, b_stack, cls_W, cls_b):
    return pl.pallas_call(
        _head_kernel,
        grid=(_NBLK,),
        in_specs=[
            pl.BlockSpec((_BLK, H), lambda i: (i, 0)),
            pl.BlockSpec((_BLK, H), lambda i: (i + _NBLK, 0)),
            pl.BlockSpec((_BLK, 1), lambda i: (i, 0)),
            pl.BlockSpec((2, H), lambda i: (0, 0)),
            pl.BlockSpec((H, C), lambda i: (0, 0)),
            pl.BlockSpec((C,), lambda i: (0,)),
        ],
        out_specs=[
            pl.BlockSpec((_BLK, H), lambda i: (i, 0)),
            pl.BlockSpec((_BLK, H), lambda i: (i, 0)),
            pl.BlockSpec((_BLK, H), lambda i: (i, 0)),
            pl.BlockSpec((_BLK, C), lambda i: (i, 0)),
        ],
        out_shape=[
            jax.ShapeDtypeStruct((N, H), _f32),
            jax.ShapeDtypeStruct((N, H), _f32),
            jax.ShapeDtypeStruct((N, H), _f32),
            jax.ShapeDtypeStruct((N, C), _f32),
        ],
    )(a_cat, a_cat, dinv_col, b_stack, cls_W, cls_b)


def kernel(x, edge_index, edge_weight, gcn1_W, gcn1_b, gcn2_W, gcn2_b,
           hgcn1_W, hgcn1_b, hgcn2_W, hgcn2_b, cls_W, cls_b):
    src, dst = edge_index[0], edge_index[1]

    pad = E3P - E3
    loop_idx = jnp.arange(N, dtype=_i32)
    src3 = jnp.concatenate([src, loop_idx, jnp.zeros((pad,), _i32)])
    dst3 = jnp.concatenate([dst, loop_idx, jnp.zeros((pad,), _i32)])
    w3 = jnp.concatenate([edge_weight, jnp.ones((N,), _f32),
                          jnp.zeros((pad,), _f32)])
    src_t = src3.reshape(NS, NB, K)
    src_t2 = jnp.stack([src_t, src_t + N])
    dst_t = dst3.reshape(NS, NB, K)

    deg_part = _sc_deg(dst.reshape(NC * NS, EPT_DEG),
                       edge_weight.reshape(NC * NS, EPT_DEG))
    xw1_cat = _tc_mm1(x, jnp.stack([gcn1_W, hgcn1_W]))
    dinv = lax.rsqrt(jnp.sum(deg_part, axis=0) + 1.0)
    dinv2d = dinv[None, :]
    dinv_col = dinv[:, None]

    c_t = _sc_escale(src3.reshape(NC * NS, EPT_SC),
                     w3.reshape(NC * NS, EPT_SC), dinv2d).reshape(NS, EPT)

    a1_cat = _sc_conv(xw1_cat, src_t2, dst_t, c_t)
    xw2_cat = _tc_mid(a1_cat, dinv_col, jnp.stack([gcn1_b, hgcn1_b]),
                      jnp.stack([gcn2_W, hgcn2_W]))

    a2_cat = _sc_conv(xw2_cat, src_t2, dst_t, c_t)
    z_gcn, z_hgcn, z, logits = _tc_head(a2_cat, dinv_col,
                                        jnp.stack([gcn2_b, hgcn2_b]),
                                        cls_W, cls_b)
    return (z_gcn, z_hgcn, z, logits)


# speedup vs baseline: 6.9531x; 1.0275x over previous
import dataclasses
import functools

import jax
import jax.numpy as jnp
from jax import lax
from jax.experimental import pallas as pl
from jax.experimental.pallas import tpu as pltpu
from jax.experimental.pallas import tpu_sc as plsc

N, E, D, H, C = 10000, 320000, 128, 128, 40
NC, NS, LN = 2, 16, 16
K = 128
E3 = E + N
NB = 168
EPT = NB * K
E3P = EPT * NS
EPT_SC = E3P // (NC * NS)
CHB = 16
ROUNDS = tuple((i * CHB, CHB) for i in range(NB // CHB)) + ((NB - 8, 8),)
RPT = N // NS
EPT_DEG = E // (NC * NS)

_f32 = jnp.float32
_i32 = jnp.int32

_vmesh = plsc.VectorSubcoreMesh(core_axis_name="c", subcore_axis_name="s")

_sc_params = pltpu.CompilerParams()
if "needs_layout_passes" in pltpu.CompilerParams.__dataclass_fields__:
    _sc_params = dataclasses.replace(_sc_params, needs_layout_passes=False)


def _deg_body(dst_hbm, w_hbm, out_hbm, acc_v, dst_v, w_v, sem):
    c = lax.axis_index("c")
    s = lax.axis_index("s")
    wid = c * NS + s
    zero16 = jnp.zeros((LN,), _f32)

    @pl.loop(0, N // LN)
    def _(i):
        acc_v[pl.ds(i * LN, LN)] = zero16

    pltpu.async_copy(dst_hbm.at[wid], dst_v, sem).wait()
    pltpu.async_copy(w_hbm.at[wid], w_v, sem).wait()

    @pl.loop(0, EPT_DEG // LN)
    def _(j):
        d16 = dst_v[pl.ds(j * LN, LN)]
        w16 = w_v[pl.ds(j * LN, LN)]
        plsc.addupdate_scatter(acc_v, [d16], w16)

    pltpu.async_copy(acc_v, out_hbm.at[wid], sem).wait()


def _sc_deg(dst_t, w_t):
    k = pl.kernel(
        _deg_body,
        out_type=jax.ShapeDtypeStruct((NC * NS, N), _f32),
        mesh=_vmesh,
        compiler_params=_sc_params,
        scratch_types=[
            pltpu.VMEM((N,), _f32),
            pltpu.VMEM((EPT_DEG,), _i32),
            pltpu.VMEM((EPT_DEG,), _f32),
            pltpu.SemaphoreType.DMA,
        ],
    )
    return k(dst_t, w_t)


def _escale_body(src_hbm, w_hbm, dinv_hbm, out_hbm, src_v, w_v, dinv_v, sem):
    c = lax.axis_index("c")
    s = lax.axis_index("s")
    wid = c * NS + s

    pltpu.async_copy(dinv_hbm.at[0], dinv_v, sem).wait()
    pltpu.async_copy(src_hbm.at[wid], src_v, sem).wait()
    pltpu.async_copy(w_hbm.at[wid], w_v, sem).wait()

    @pl.loop(0, EPT_SC // LN, unroll=4)
    def _(m):
        slc = pl.ds(m * LN, LN)
        w_v[slc] = w_v[slc] * plsc.load_gather(dinv_v, [src_v[slc]])

    pltpu.async_copy(w_v, out_hbm.at[wid], sem).wait()


def _sc_escale(src_flat, w_flat, dinv2d):
    k = pl.kernel(
        _escale_body,
        out_type=jax.ShapeDtypeStruct((NC * NS, EPT_SC), _f32),
        mesh=_vmesh,
        compiler_params=_sc_params,
        scratch_types=[
            pltpu.VMEM((EPT_SC,), _i32),
            pltpu.VMEM((EPT_SC,), _f32),
            pltpu.VMEM((N,), _f32),
            pltpu.SemaphoreType.DMA,
        ],
    )
    return k(src_flat, w_flat, dinv2d)


def _conv_body(xw_hbm, src_hbm, dst_hbm, c_hbm, out_hbm,
               acc_sh, src0_v, dst0_v, c0_v, src1_v, dst1_v, c1_v,
               rows0_v, rows1_v, sem, ssem0, ssem1, gsem0, gsem1):
    c = lax.axis_index("c")
    s = lax.axis_index("s")

    zero16 = jnp.zeros((LN,), _f32)

    @pl.loop(0, K)
    def _(r):
        for cc in range(H // LN):
            rows0_v[r, pl.ds(cc * LN, LN)] = zero16

    r0 = pl.multiple_of(s * 632, 8)
    last_off = pl.multiple_of(jnp.where(s == NS - 1, 392, 504), 8)
    for off in (0, K, 2 * K, 3 * K):
        pltpu.async_copy(rows0_v, acc_sh.at[pl.ds(r0 + off, K)], sem).wait()
    pltpu.async_copy(rows0_v, acc_sh.at[pl.ds(r0 + last_off, K)], sem).wait()

    plsc.subcore_barrier()

    bufs = ((rows0_v, gsem0), (rows1_v, gsem1))
    sets = ((src0_v, dst0_v, c0_v, ssem0), (src1_v, dst1_v, c1_v, ssem1))

    def stage(st, bstart, nblk):
        srcb, dstb, cb, ssem = st
        pltpu.async_copy(src_hbm.at[c, s, pl.ds(bstart, nblk)],
                         srcb.at[pl.ds(0, nblk)], ssem)
        pltpu.async_copy(dst_hbm.at[s, pl.ds(bstart, nblk)],
                         dstb.at[pl.ds(0, nblk)], ssem)
        pltpu.async_copy(c_hbm.at[s, pl.ds(bstart * K, nblk * K)],
                         cb.at[pl.ds(0, nblk * K)], ssem)

    def stage_wait(st, nblk):
        srcb, dstb, cb, ssem = st
        pltpu.make_async_copy(src_hbm.at[c, s, pl.ds(0, nblk)],
                              srcb.at[pl.ds(0, nblk)], ssem).wait()
        pltpu.make_async_copy(dst_hbm.at[s, pl.ds(0, nblk)],
                              dstb.at[pl.ds(0, nblk)], ssem).wait()
        pltpu.make_async_copy(c_hbm.at[s, pl.ds(0, nblk * K)],
                              cb.at[pl.ds(0, nblk * K)], ssem).wait()

    def scale_rows(rows_v, cb, blk):
        @pl.loop(0, K, unroll=2)
        def _(j):
            scale = plsc.load_gather(cb, [jnp.zeros((LN,), _i32) + blk * K + j])
            for cc in range(H // LN):
                slc = (j, pl.ds(cc * LN, LN))
                rows_v[slc] = rows_v[slc] * scale

    nrounds = len(ROUNDS)
    stage(sets[0], ROUNDS[0][0], ROUNDS[0][1])
    stage(sets[1], ROUNDS[1][0], ROUNDS[1][1])
    stage_wait(sets[0], ROUNDS[0][1])
    pltpu.async_copy(xw_hbm.at[src0_v.at[0]], rows0_v, gsem0)
    pltpu.async_copy(xw_hbm.at[src0_v.at[1]], rows1_v, gsem1)

    for r, (bstart, nblk) in enumerate(ROUNDS):
        cur = sets[r % 2]
        nxt = sets[(r + 1) % 2]
        csrc, cdst, cc_v, _ = cur

        @pl.loop(0, nblk - 2, step=2)
        def _(b):
            for par, (rows_v, gsem) in enumerate(bufs):
                blk = b + par
                pltpu.make_async_copy(xw_hbm.at[csrc.at[0]], rows_v, gsem).wait()
                scale_rows(rows_v, cc_v, blk)
                pltpu.sync_copy(rows_v, acc_sh.at[cdst.at[blk]], add=True)
                pltpu.async_copy(xw_hbm.at[csrc.at[blk + 2]], rows_v, gsem)

        if r + 1 < nrounds:
            stage_wait(nxt, ROUNDS[r + 1][1])
        for par, (rows_v, gsem) in enumerate(bufs):
            blk = nblk - 2 + par
            pltpu.make_async_copy(xw_hbm.at[csrc.at[0]], rows_v, gsem).wait()
            scale_rows(rows_v, cc_v, blk)
            pltpu.sync_copy(rows_v, acc_sh.at[cdst.at[blk]], add=True)
            if r + 1 < nrounds:
                pltpu.async_copy(xw_hbm.at[nxt[0].at[par]], rows_v, gsem)
        if r + 2 < nrounds:
            stage(cur, ROUNDS[r + 2][0], ROUNDS[r + 2][1])

    plsc.subcore_barrier()

    for off in (0, K, 2 * K, 3 * K, None):
        o = r0 + (last_off if off is None else off)
        oo = pl.multiple_of(c * N + o, 8)
        pltpu.sync_copy(acc_sh.at[pl.ds(o, K)], out_hbm.at[pl.ds(oo, K)])


def _sc_conv(xw_cat, src_t2, dst_t, c_t):
    k = pl.kernel(
        _conv_body,
        out_type=jax.ShapeDtypeStruct((2 * N, H), _f32),
        mesh=_vmesh,
        compiler_params=_sc_params,
        scratch_types=[
            pltpu.VMEM_SHARED((N, H), _f32),
            pltpu.VMEM((CHB, K), _i32),
            pltpu.VMEM((CHB, K), _i32),
            pltpu.VMEM((CHB * K,), _f32),
            pltpu.VMEM((CHB, K), _i32),
            pltpu.VMEM((CHB, K), _i32),
            pltpu.VMEM((CHB * K,), _f32),
            pltpu.VMEM((K, H), _f32),
            pltpu.VMEM((K, H), _f32),
            pltpu.SemaphoreType.DMA,
            pltpu.SemaphoreType.DMA,
            pltpu.SemaphoreType.DMA,
            pltpu.SemaphoreType.DMA,
            pltpu.SemaphoreType.DMA,
        ],
    )
    return k(xw_cat, src_t2, dst_t, c_t)


_BLK = 1000
_NBLK = N // _BLK


def _mm1_kernel(x_ref, w_ref, o_ref):
    o_ref[...] = jnp.dot(x_ref[...], w_ref[0], preferred_element_type=_f32)


def _tc_mm1(x, W_stack):
    return pl.pallas_call(
        _mm1_kernel,
        grid=(2 * _NBLK,),
        in_specs=[
            pl.BlockSpec((_BLK, D), lambda i: (i % _NBLK, 0)),
            pl.BlockSpec((1, D, H), lambda i: (i // _NBLK, 0, 0)),
        ],
        out_specs=pl.BlockSpec((_BLK, H), lambda i: (i, 0)),
        out_shape=jax.ShapeDtypeStruct((2 * N, H), _f32),
    )(x, W_stack)


def _mid_kernel(a_ref, dv_ref, b_ref, w_ref, o_ref):
    h = jax.nn.relu(a_ref[...] * dv_ref[...] + b_ref[0])
    o_ref[...] = jnp.dot(h, w_ref[0], preferred_element_type=_f32)


def _tc_mid(a_cat, dinv_col, b_stack, W_stack):
    return pl.pallas_call(
        _mid_kernel,
        grid=(2 * _NBLK,),
        in_specs=[
            pl.BlockSpec((_BLK, H), lambda i: (i, 0)),
            pl.BlockSpec((_BLK, 1), lambda i: (i % _NBLK, 0)),
            pl.BlockSpec((1, 1, H), lambda i: (i // _NBLK, 0, 0)),
            pl.BlockSpec((1, H, H), lambda i: (i // _NBLK, 0, 0)),
        ],
        out_specs=pl.BlockSpec((_BLK, H), lambda i: (i, 0)),
        out_shape=jax.ShapeDtypeStruct((2 * N, H), _f32),
    )(a_cat, dinv_col, b_stack.reshape(2, 1, H), W_stack)


def _head_kernel(ag_ref, ah_ref, dv_ref, b_ref, w_ref, cb_ref,
                 zg_ref, zh_ref, z_ref, lg_ref):
    dv = dv_ref[...]
    hg = ag_ref[...] * dv + b_ref[0, :][None, :]
    hh = ah_ref[...] * dv + b_ref[1, :][None, :]

    def l2n(v):
        nrm = jnp.sqrt(jnp.sum(v * v, axis=1, keepdims=True))
        return v / jnp.maximum(nrm, 1e-12)

    zg = l2n(hg)
    zh = l2n(hh)
    z = l2n(0.6 * zg + 0.4 * zh)
    zg_ref[...] = zg
    zh_ref[...] = zh
    z_ref[...] = z
    lg_ref[...] = jnp.dot(z, w_ref[...], preferred_element_type=_f32) + cb_ref[...]


def _tc_head(a_cat, dinv_col, b_stack, cls_W, cls_b):
    return pl.pallas_call(
        _head_kernel,
        grid=(_NBLK,),
        in_specs=[
            pl.BlockSpec((_BLK, H), lambda i: (i, 0)),
            pl.BlockSpec((_BLK, H), lambda i: (i + _NBLK, 0)),
            pl.BlockSpec((_BLK, 1), lambda i: (i, 0)),
            pl.BlockSpec((2, H), lambda i: (0, 0)),
            pl.BlockSpec((H, C), lambda i: (0, 0)),
            pl.BlockSpec((C,), lambda i: (0,)),
        ],
        out_specs=[
            pl.BlockSpec((_BLK, H), lambda i: (i, 0)),
            pl.BlockSpec((_BLK, H), lambda i: (i, 0)),
            pl.BlockSpec((_BLK, H), lambda i: (i, 0)),
            pl.BlockSpec((_BLK, C), lambda i: (i, 0)),
        ],
        out_shape=[
            jax.ShapeDtypeStruct((N, H), _f32),
            jax.ShapeDtypeStruct((N, H), _f32),
            jax.ShapeDtypeStruct((N, H), _f32),
            jax.ShapeDtypeStruct((N, C), _f32),
        ],
    )(a_cat, a_cat, dinv_col, b_stack, cls_W, cls_b)


def kernel(x, edge_index, edge_weight, gcn1_W, gcn1_b, gcn2_W, gcn2_b,
           hgcn1_W, hgcn1_b, hgcn2_W, hgcn2_b, cls_W, cls_b):
    src, dst = edge_index[0], edge_index[1]

    pad = E3P - E3
    loop_idx = jnp.arange(N, dtype=_i32)
    src3 = jnp.concatenate([src, loop_idx, jnp.zeros((pad,), _i32)])
    dst3 = jnp.concatenate([dst, loop_idx, jnp.zeros((pad,), _i32)])
    w3 = jnp.concatenate([edge_weight, jnp.ones((N,), _f32),
                          jnp.zeros((pad,), _f32)])
    src_t = src3.reshape(NS, NB, K)
    src_t2 = jnp.stack([src_t, src_t + N])
    dst_t = dst3.reshape(NS, NB, K)

    deg_part = _sc_deg(dst.reshape(NC * NS, EPT_DEG),
                       edge_weight.reshape(NC * NS, EPT_DEG))
    xw1_cat = _tc_mm1(x, jnp.stack([gcn1_W, hgcn1_W]))
    dinv = lax.rsqrt(jnp.sum(deg_part, axis=0) + 1.0)
    dinv2d = dinv[None, :]
    dinv_col = dinv[:, None]

    c_t = _sc_escale(src3.reshape(NC * NS, EPT_SC),
                     w3.reshape(NC * NS, EPT_SC), dinv2d).reshape(NS, EPT)

    a1_cat = _sc_conv(xw1_cat, src_t2, dst_t, c_t)
    xw2_cat = _tc_mid(a1_cat, dinv_col, jnp.stack([gcn1_b, hgcn1_b]),
                      jnp.stack([gcn2_W, hgcn2_W]))

    a2_cat = _sc_conv(xw2_cat, src_t2, dst_t, c_t)
    z_gcn, z_hgcn, z, logits = _tc_head(a2_cat, dinv_col,
                                        jnp.stack([gcn2_b, hgcn2_b]),
                                        cls_W, cls_b)
    return (z_gcn, z_hgcn, z, logits)


# speedup vs baseline: 6.9569x; 1.0005x over previous
import dataclasses

import jax
import jax.numpy as jnp
from jax import lax
from jax.experimental import pallas as pl
from jax.experimental.pallas import tpu as pltpu
from jax.experimental.pallas import tpu_sc as plsc

N, E, D, H, C = 10000, 320000, 128, 128, 40
NC, NS, LN = 2, 16, 16
K = 128
E3 = E + N
NB = 168
EPT = NB * K
E3P = EPT * NS
EPT_SC = E3P // (NC * NS)
CHB = 16
ROUNDS = tuple((i * CHB, CHB) for i in range(NB // CHB)) + ((NB - 8, 8),)
EPT_DEG = E // (NC * NS)

_f32 = jnp.float32
_i32 = jnp.int32

_vmesh = plsc.VectorSubcoreMesh(core_axis_name="c", subcore_axis_name="s")

_sc_params = pltpu.CompilerParams()
if "needs_layout_passes" in pltpu.CompilerParams.__dataclass_fields__:
    _sc_params = dataclasses.replace(_sc_params, needs_layout_passes=False)


def _deg_body(dst_hbm, w_hbm, out_hbm, acc_v, dst_v, w_v, sem):
    c = lax.axis_index("c")
    s = lax.axis_index("s")
    wid = c * NS + s
    zero16 = jnp.zeros((LN,), _f32)

    @pl.loop(0, N // LN)
    def _(i):
        acc_v[pl.ds(i * LN, LN)] = zero16

    pltpu.async_copy(dst_hbm.at[wid], dst_v, sem).wait()
    pltpu.async_copy(w_hbm.at[wid], w_v, sem).wait()

    @pl.loop(0, EPT_DEG // LN)
    def _(j):
        d16 = dst_v[pl.ds(j * LN, LN)]
        w16 = w_v[pl.ds(j * LN, LN)]
        plsc.addupdate_scatter(acc_v, [d16], w16)

    pltpu.async_copy(acc_v, out_hbm.at[wid], sem).wait()


def _sc_deg(dst_t, w_t):
    k = pl.kernel(
        _deg_body,
        out_type=jax.ShapeDtypeStruct((NC * NS, N), _f32),
        mesh=_vmesh,
        compiler_params=_sc_params,
        scratch_types=[
            pltpu.VMEM((N,), _f32),
            pltpu.VMEM((EPT_DEG,), _i32),
            pltpu.VMEM((EPT_DEG,), _f32),
            pltpu.SemaphoreType.DMA,
        ],
    )
    return k(dst_t, w_t)


def _escale_body(src_hbm, w_hbm, dinv_hbm, out_hbm, src_v, w_v, dinv_v, sem):
    c = lax.axis_index("c")
    s = lax.axis_index("s")
    wid = c * NS + s

    pltpu.async_copy(dinv_hbm.at[0], dinv_v, sem).wait()
    pltpu.async_copy(src_hbm.at[wid], src_v, sem).wait()
    pltpu.async_copy(w_hbm.at[wid], w_v, sem).wait()

    @pl.loop(0, EPT_SC // LN, unroll=4)
    def _(m):
        slc = pl.ds(m * LN, LN)
        w_v[slc] = w_v[slc] * plsc.load_gather(dinv_v, [src_v[slc]])

    pltpu.async_copy(w_v, out_hbm.at[wid], sem).wait()


def _sc_escale(src_flat, w_flat, dinv2d):
    k = pl.kernel(
        _escale_body,
        out_type=jax.ShapeDtypeStruct((NC * NS, EPT_SC), _f32),
        mesh=_vmesh,
        compiler_params=_sc_params,
        scratch_types=[
            pltpu.VMEM((EPT_SC,), _i32),
            pltpu.VMEM((EPT_SC,), _f32),
            pltpu.VMEM((N,), _f32),
            pltpu.SemaphoreType.DMA,
        ],
    )
    return k(src_flat, w_flat, dinv2d)


def _conv_body(xw_hbm, src_hbm, dst_hbm, c_hbm, out_hbm,
               acc_sh, src0_v, dst0_v, c0_v, src1_v, dst1_v, c1_v,
               rows0_v, rows1_v, sem, ssem0, ssem1, gsem0, gsem1):
    c = lax.axis_index("c")
    s = lax.axis_index("s")

    zero16 = jnp.zeros((LN,), _f32)

    @pl.loop(0, K)
    def _(r):
        for cc in range(H // LN):
            rows0_v[r, pl.ds(cc * LN, LN)] = zero16

    r0 = pl.multiple_of(s * 632, 8)
    last_off = pl.multiple_of(jnp.where(s == NS - 1, 392, 504), 8)
    for off in (0, K, 2 * K, 3 * K):
        pltpu.async_copy(rows0_v, acc_sh.at[pl.ds(r0 + off, K)], sem).wait()
    pltpu.async_copy(rows0_v, acc_sh.at[pl.ds(r0 + last_off, K)], sem).wait()

    plsc.subcore_barrier()

    bufs = ((rows0_v, gsem0), (rows1_v, gsem1))
    sets = ((src0_v, dst0_v, c0_v, ssem0), (src1_v, dst1_v, c1_v, ssem1))

    def stage(st, bstart, nblk):
        srcb, dstb, cb, ssem = st
        pltpu.async_copy(src_hbm.at[c, s, pl.ds(bstart, nblk)],
                         srcb.at[pl.ds(0, nblk)], ssem)
        pltpu.async_copy(dst_hbm.at[s, pl.ds(bstart, nblk)],
                         dstb.at[pl.ds(0, nblk)], ssem)
        pltpu.async_copy(c_hbm.at[s, pl.ds(bstart * K, nblk * K)],
                         cb.at[pl.ds(0, nblk * K)], ssem)

    def stage_wait(st, nblk):
        srcb, dstb, cb, ssem = st
        pltpu.make_async_copy(src_hbm.at[c, s, pl.ds(0, nblk)],
                              srcb.at[pl.ds(0, nblk)], ssem).wait()
        pltpu.make_async_copy(dst_hbm.at[s, pl.ds(0, nblk)],
                              dstb.at[pl.ds(0, nblk)], ssem).wait()
        pltpu.make_async_copy(c_hbm.at[s, pl.ds(0, nblk * K)],
                              cb.at[pl.ds(0, nblk * K)], ssem).wait()

    def scale_rows(rows_v, cb, blk):
        @pl.loop(0, K, unroll=2)
        def _(j):
            scale = plsc.load_gather(cb, [jnp.zeros((LN,), _i32) + blk * K + j])
            for cc in range(H // LN):
                slc = (j, pl.ds(cc * LN, LN))
                rows_v[slc] = rows_v[slc] * scale

    nrounds = len(ROUNDS)
    stage(sets[0], ROUNDS[0][0], ROUNDS[0][1])
    stage(sets[1], ROUNDS[1][0], ROUNDS[1][1])
    stage_wait(sets[0], ROUNDS[0][1])
    pltpu.async_copy(xw_hbm.at[src0_v.at[0]], rows0_v, gsem0)
    pltpu.async_copy(xw_hbm.at[src0_v.at[1]], rows1_v, gsem1)

    for r, (bstart, nblk) in enumerate(ROUNDS):
        cur = sets[r % 2]
        nxt = sets[(r + 1) % 2]
        csrc, cdst, cc_v, _ = cur

        @pl.loop(0, nblk - 2, step=2)
        def _(b):
            for par, (rows_v, gsem) in enumerate(bufs):
                blk = b + par
                pltpu.make_async_copy(xw_hbm.at[csrc.at[0]], rows_v, gsem).wait()
                scale_rows(rows_v, cc_v, blk)
                pltpu.sync_copy(rows_v, acc_sh.at[cdst.at[blk]], add=True)
                pltpu.async_copy(xw_hbm.at[csrc.at[blk + 2]], rows_v, gsem)

        if r + 1 < nrounds:
            stage_wait(nxt, ROUNDS[r + 1][1])
        for par, (rows_v, gsem) in enumerate(bufs):
            blk = nblk - 2 + par
            pltpu.make_async_copy(xw_hbm.at[csrc.at[0]], rows_v, gsem).wait()
            scale_rows(rows_v, cc_v, blk)
            pltpu.sync_copy(rows_v, acc_sh.at[cdst.at[blk]], add=True)
            if r + 1 < nrounds:
                pltpu.async_copy(xw_hbm.at[nxt[0].at[par]], rows_v, gsem)
        if r + 2 < nrounds:
            stage(cur, ROUNDS[r + 2][0], ROUNDS[r + 2][1])

    plsc.subcore_barrier()

    for off in (0, K, 2 * K, 3 * K, None):
        o = r0 + (last_off if off is None else off)
        oo = pl.multiple_of(c * N + o, 8)
        pltpu.sync_copy(acc_sh.at[pl.ds(o, K)], out_hbm.at[pl.ds(oo, K)])


def _sc_conv(xw_cat, src_t2, dst_t, c_t):
    k = pl.kernel(
        _conv_body,
        out_type=jax.ShapeDtypeStruct((2 * N, H), _f32),
        mesh=_vmesh,
        compiler_params=_sc_params,
        scratch_types=[
            pltpu.VMEM_SHARED((N, H), _f32),
            pltpu.VMEM((CHB, K), _i32),
            pltpu.VMEM((CHB, K), _i32),
            pltpu.VMEM((CHB * K,), _f32),
            pltpu.VMEM((CHB, K), _i32),
            pltpu.VMEM((CHB, K), _i32),
            pltpu.VMEM((CHB * K,), _f32),
            pltpu.VMEM((K, H), _f32),
            pltpu.VMEM((K, H), _f32),
            pltpu.SemaphoreType.DMA,
            pltpu.SemaphoreType.DMA,
            pltpu.SemaphoreType.DMA,
            pltpu.SemaphoreType.DMA,
            pltpu.SemaphoreType.DMA,
        ],
    )
    return k(xw_cat, src_t2, dst_t, c_t)


_BLK = 1000
_NBLK = N // _BLK


def _mm1_kernel(x_ref, w_ref, o_ref):
    o_ref[...] = jnp.dot(x_ref[...], w_ref[0], preferred_element_type=_f32)


def _tc_mm1(x, W_stack):
    return pl.pallas_call(
        _mm1_kernel,
        grid=(2 * _NBLK,),
        in_specs=[
            pl.BlockSpec((_BLK, D), lambda i: (i % _NBLK, 0)),
            pl.BlockSpec((1, D, H), lambda i: (i // _NBLK, 0, 0)),
        ],
        out_specs=pl.BlockSpec((_BLK, H), lambda i: (i, 0)),
        out_shape=jax.ShapeDtypeStruct((2 * N, H), _f32),
    )(x, W_stack)


def _mid_kernel(a_ref, dv_ref, b_ref, w_ref, o_ref):
    h = jax.nn.relu(a_ref[...] * dv_ref[...] + b_ref[0])
    o_ref[...] = jnp.dot(h, w_ref[0], preferred_element_type=_f32)


def _tc_mid(a_cat, dinv_col, b_stack, W_stack):
    return pl.pallas_call(
        _mid_kernel,
        grid=(2 * _NBLK,),
        in_specs=[
            pl.BlockSpec((_BLK, H), lambda i: (i, 0)),
            pl.BlockSpec((_BLK, 1), lambda i: (i % _NBLK, 0)),
            pl.BlockSpec((1, 1, H), lambda i: (i // _NBLK, 0, 0)),
            pl.BlockSpec((1, H, H), lambda i: (i // _NBLK, 0, 0)),
        ],
        out_specs=pl.BlockSpec((_BLK, H), lambda i: (i, 0)),
        out_shape=jax.ShapeDtypeStruct((2 * N, H), _f32),
    )(a_cat, dinv_col, b_stack.reshape(2, 1, H), W_stack)


def _head_kernel(ag_ref, ah_ref, dv_ref, b_ref, w_ref, cb_ref,
                 zg_ref, zh_ref, z_ref, lg_ref):
    dv = dv_ref[...]
    hg = ag_ref[...] * dv + b_ref[0, :][None, :]
    hh = ah_ref[...] * dv + b_ref[1, :][None, :]

    def l2n(v):
        nrm = jnp.sqrt(jnp.sum(v * v, axis=1, keepdims=True))
        return v / jnp.maximum(nrm, 1e-12)

    zg = l2n(hg)
    zh = l2n(hh)
    z = l2n(0.6 * zg + 0.4 * zh)
    zg_ref[...] = zg
    zh_ref[...] = zh
    z_ref[...] = z
    lg_ref[...] = jnp.dot(z, w_ref[...], preferred_element_type=_f32) + cb_ref[...]


def _tc_head(a_cat, dinv_col, b_stack, cls_W, cls_b):
    return pl.pallas_call(
        _head_kernel,
        grid=(_NBLK,),
        in_specs=[
            pl.BlockSpec((_BLK, H), lambda i: (i, 0)),
            pl.BlockSpec((_BLK, H), lambda i: (i + _NBLK, 0)),
            pl.BlockSpec((_BLK, 1), lambda i: (i, 0)),
            pl.BlockSpec((2, H), lambda i: (0, 0)),
            pl.BlockSpec((H, C), lambda i: (0, 0)),
            pl.BlockSpec((C,), lambda i: (0,)),
        ],
        out_specs=[
            pl.BlockSpec((_BLK, H), lambda i: (i, 0)),
            pl.BlockSpec((_BLK, H), lambda i: (i, 0)),
            pl.BlockSpec((_BLK, H), lambda i: (i, 0)),
            pl.BlockSpec((_BLK, C), lambda i: (i, 0)),
        ],
        out_shape=[
            jax.ShapeDtypeStruct((N, H), _f32),
            jax.ShapeDtypeStruct((N, H), _f32),
            jax.ShapeDtypeStruct((N, H), _f32),
            jax.ShapeDtypeStruct((N, C), _f32),
        ],
    )(a_cat, a_cat, dinv_col, b_stack, cls_W, cls_b)


def kernel(x, edge_index, edge_weight, gcn1_W, gcn1_b, gcn2_W, gcn2_b,
           hgcn1_W, hgcn1_b, hgcn2_W, hgcn2_b, cls_W, cls_b):
    src, dst = edge_index[0], edge_index[1]

    pad = E3P - E3
    loop_idx = jnp.arange(N, dtype=_i32)
    src3 = jnp.concatenate([src, loop_idx, jnp.zeros((pad,), _i32)])
    dst3 = jnp.concatenate([dst, loop_idx, jnp.zeros((pad,), _i32)])
    w3 = jnp.concatenate([edge_weight, jnp.ones((N,), _f32),
                          jnp.zeros((pad,), _f32)])
    src_t = src3.reshape(NS, NB, K)
    src_t2 = jnp.stack([src_t, src_t + N])
    dst_t = dst3.reshape(NS, NB, K)

    deg_part = _sc_deg(dst.reshape(NC * NS, EPT_DEG),
                       edge_weight.reshape(NC * NS, EPT_DEG))
    xw1_cat = _tc_mm1(x, jnp.stack([gcn1_W, hgcn1_W]))
    dinv = lax.rsqrt(jnp.sum(deg_part, axis=0) + 1.0)
    dinv2d = dinv[None, :]
    dinv_col = dinv[:, None]

    c_t = _sc_escale(src3.reshape(NC * NS, EPT_SC),
                     w3.reshape(NC * NS, EPT_SC), dinv2d).reshape(NS, EPT)

    a1_cat = _sc_conv(xw1_cat, src_t2, dst_t, c_t)
    xw2_cat = _tc_mid(a1_cat, dinv_col, jnp.stack([gcn1_b, hgcn1_b]),
                      jnp.stack([gcn2_W, hgcn2_W]))

    a2_cat = _sc_conv(xw2_cat, src_t2, dst_t, c_t)
    z_gcn, z_hgcn, z, logits = _tc_head(a2_cat, dinv_col,
                                        jnp.stack([gcn2_b, hgcn2_b]),
                                        cls_W, cls_b)
    return (z_gcn, z_hgcn, z, logits)
